# baseline (speedup 1.0000x reference)
import jax
import jax.numpy as jnp
from jax import lax
from jax.experimental import pallas as pl
from jax.experimental.pallas import tpu as pltpu, tpu_sc as plsc

N = 1048576
W_IN = 8
W_OUT = 4
P = 16

NC = 2
NS = 16
NW = NC * NS
LANES = 16
BLK = 128

NBLK = N // BLK
BLK_PER_W = NBLK // NW
CHB = 16
NCHUNK = BLK_PER_W // CHB
LUT_SIZE = 256 * W_OUT


def _body(in_hbm, addr_hbm, res_hbm, out_hbm,
          buf0, buf1, buf2, lut, addr_v, res_v,
          in_sem0, in_sem1, in_sem2, out_sem0, out_sem1, out_sem2):
    wid = lax.axis_index("s") * NC + lax.axis_index("c")
    blk_base = wid * BLK_PER_W

    pltpu.sync_copy(addr_hbm, addr_v)
    pltpu.sync_copy(res_hbm, res_v)

    nanv = jnp.full((LANES,), jnp.nan, dtype=jnp.float32)

    def init_body(k, c):
        lut[pl.ds(k * LANES, LANES)] = nanv
        return c

    lax.fori_loop(0, LUT_SIZE // LANES, init_body, 0)

    code_p = addr_v[pl.ds(0, LANES)]
    for j in range(1, W_IN):
        code_p = code_p + (addr_v[pl.ds(j * LANES, LANES)] << j)
    for c in range(W_OUT):
        plsc.store_scatter(lut, [code_p * W_OUT + c],
                           res_v[pl.ds(c * LANES, LANES)])

    GROUPS = BLK // LANES

    def compute(buf):
        @plsc.parallel_loop(0, CHB * GROUPS, step=1, unroll=1)
        def body(k):
            b = k >> 3
            goff = (k & 7) * LANES
            x = [buf[b, c, pl.ds(goff, LANES)] for c in range(W_IN)]
            t = [x[c] * float(1 << c) for c in range(1, W_IN)]
            s01 = x[0] + t[0]
            s23 = t[1] + t[2]
            s45 = t[3] + t[4]
            s67 = t[5] + t[6]
            code = (s01 + s23) + (s45 + s67)
            base4 = code.astype(jnp.int32) * W_OUT
            for c in range(W_OUT):
                e = plsc.load_gather(lut, [base4 + c])
                buf[b, c, pl.ds(goff, LANES)] = jnp.where(e == e, e, x[c])

    bufs = (buf0, buf1, buf2)
    in_sems = (in_sem0, in_sem1, in_sem2)
    out_sems = (out_sem0, out_sem1, out_sem2)
    NBUF = 3

    def in_copy(g):
        return pltpu.make_async_copy(
            in_hbm.at[pl.ds(blk_base + g * CHB, CHB)],
            bufs[g % NBUF], in_sems[g % NBUF])

    def out_copy(g):
        return pltpu.make_async_copy(
            bufs[g % NBUF],
            out_hbm.at[pl.ds(blk_base + g * CHB, CHB)],
            out_sems[g % NBUF])

    in_copy(0).start()
    in_copy(1).start()
    for g in range(NCHUNK):
        if g + 2 < NCHUNK:
            if g >= 1:
                out_copy(g - 1).wait()
            in_copy(g + 2).start()
        in_copy(g).wait()
        compute(bufs[g % NBUF])
        out_copy(g).start()
    for g in range(max(0, NCHUNK - 3), NCHUNK):
        out_copy(g).wait()


@jax.jit
def _run(tensor, addresses, results):
    in3 = tensor.reshape(NBLK, BLK, W_IN).transpose(0, 2, 1)
    addr1 = addresses.astype(jnp.int32).T.reshape(-1)
    res1 = results.astype(jnp.float32).T.reshape(-1)
    kfn = pl.kernel(
        _body,
        out_type=jax.ShapeDtypeStruct((NBLK, W_IN, BLK), jnp.float32),
        mesh=plsc.VectorSubcoreMesh(core_axis_name="c", subcore_axis_name="s"),
        compiler_params=pltpu.CompilerParams(
            needs_layout_passes=False, use_tc_tiling_on_sc=False),
        scratch_types=[
            pltpu.VMEM((CHB, W_IN, BLK), jnp.float32),
            pltpu.VMEM((CHB, W_IN, BLK), jnp.float32),
            pltpu.VMEM((CHB, W_IN, BLK), jnp.float32),
            pltpu.VMEM((LUT_SIZE,), jnp.float32),
            pltpu.VMEM((W_IN * P,), jnp.int32),
            pltpu.VMEM((W_OUT * P,), jnp.float32),
            pltpu.SemaphoreType.DMA,
            pltpu.SemaphoreType.DMA,
            pltpu.SemaphoreType.DMA,
            pltpu.SemaphoreType.DMA,
            pltpu.SemaphoreType.DMA,
            pltpu.SemaphoreType.DMA,
        ],
    )
    out3 = kfn(in3, addr1, res1)
    return out3.transpose(0, 2, 1).reshape(N, W_IN)


def kernel(tensor, addresses, results):
    return _run(tensor, addresses, results)


# speedup vs baseline: 1.0979x; 1.0979x over previous
import jax
import jax.numpy as jnp
from jax import lax
from jax.experimental import pallas as pl
from jax.experimental.pallas import tpu as pltpu, tpu_sc as plsc

N = 1048576
W_IN = 8
W_OUT = 4
P = 16

NC = 2
NS = 16
NW = NC * NS
LANES = 16
BLK = 128

NBLK = N // BLK
BLK_PER_W = NBLK // NW
CHB = 32
NCHUNK = BLK_PER_W // CHB
LUT_SIZE = 256 * W_OUT


def _body(in_hbm, addr_hbm, res_hbm, out_hbm,
          buf0, buf1, buf2, lut, addr_v, res_v,
          in_sem0, in_sem1, in_sem2, out_sem0, out_sem1, out_sem2):
    wid = lax.axis_index("s") * NC + lax.axis_index("c")
    blk_base = wid * BLK_PER_W

    bufs = (buf0, buf1, buf2)
    in_sems = (in_sem0, in_sem1, in_sem2)
    out_sems = (out_sem0, out_sem1, out_sem2)
    NBUF = 3

    def in_copy(g):
        return pltpu.make_async_copy(
            in_hbm.at[pl.ds(blk_base + g * CHB, CHB)],
            bufs[g % NBUF], in_sems[g % NBUF])

    def out_copy(g):
        return pltpu.make_async_copy(
            bufs[g % NBUF],
            out_hbm.at[pl.ds(blk_base + g * CHB, CHB)],
            out_sems[g % NBUF])

    in_copy(0).start()
    in_copy(1).start()

    pltpu.sync_copy(addr_hbm, addr_v)
    pltpu.sync_copy(res_hbm, res_v)

    nanv = jnp.full((LANES,), jnp.nan, dtype=jnp.float32)

    def init_body(k, c):
        lut[pl.ds(k * LANES, LANES)] = nanv
        return c

    lax.fori_loop(0, LUT_SIZE // LANES, init_body, 0)

    code_p = addr_v[pl.ds(0, LANES)]
    for j in range(1, W_IN):
        code_p = code_p + (addr_v[pl.ds(j * LANES, LANES)] << j)
    for c in range(W_OUT):
        plsc.store_scatter(lut, [code_p * W_OUT + c],
                           res_v[pl.ds(c * LANES, LANES)])

    GROUPS = BLK // LANES

    def compute(buf):
        @plsc.parallel_loop(0, CHB * GROUPS, step=1, unroll=1)
        def body(k):
            b = k >> 3
            goff = (k & 7) * LANES
            x = [buf[b, c, pl.ds(goff, LANES)] for c in range(W_IN)]
            t = [x[c] * float(1 << c) for c in range(1, W_IN)]
            s01 = x[0] + t[0]
            s23 = t[1] + t[2]
            s45 = t[3] + t[4]
            s67 = t[5] + t[6]
            code = (s01 + s23) + (s45 + s67)
            base4 = code.astype(jnp.int32) * W_OUT
            for c in range(W_OUT):
                e = plsc.load_gather(lut, [base4 + c])
                buf[b, c, pl.ds(goff, LANES)] = jnp.where(e == e, e, x[c])

    for g in range(NCHUNK):
        if g + 2 < NCHUNK:
            if g >= 1:
                out_copy(g - 1).wait()
            in_copy(g + 2).start()
        in_copy(g).wait()
        compute(bufs[g % NBUF])
        out_copy(g).start()
    for g in range(max(0, NCHUNK - 3), NCHUNK):
        out_copy(g).wait()


@jax.jit
def _run(tensor, addresses, results):
    in3 = tensor.reshape(NBLK, BLK, W_IN).transpose(0, 2, 1)
    addr1 = addresses.astype(jnp.int32).T.reshape(-1)
    res1 = results.astype(jnp.float32).T.reshape(-1)
    kfn = pl.kernel(
        _body,
        out_type=jax.ShapeDtypeStruct((NBLK, W_IN, BLK), jnp.float32),
        mesh=plsc.VectorSubcoreMesh(core_axis_name="c", subcore_axis_name="s"),
        compiler_params=pltpu.CompilerParams(
            needs_layout_passes=False, use_tc_tiling_on_sc=False),
        scratch_types=[
            pltpu.VMEM((CHB, W_IN, BLK), jnp.float32),
            pltpu.VMEM((CHB, W_IN, BLK), jnp.float32),
            pltpu.VMEM((CHB, W_IN, BLK), jnp.float32),
            pltpu.VMEM((LUT_SIZE,), jnp.float32),
            pltpu.VMEM((W_IN * P,), jnp.int32),
            pltpu.VMEM((W_OUT * P,), jnp.float32),
            pltpu.SemaphoreType.DMA,
            pltpu.SemaphoreType.DMA,
            pltpu.SemaphoreType.DMA,
            pltpu.SemaphoreType.DMA,
            pltpu.SemaphoreType.DMA,
            pltpu.SemaphoreType.DMA,
        ],
    )
    out3 = kfn(in3, addr1, res1)
    return out3.transpose(0, 2, 1).reshape(N, W_IN)


def kernel(tensor, addresses, results):
    return _run(tensor, addresses, results)
